# baseline (speedup 1.0000x reference)
import jax
import jax.numpy as jnp
from jax import lax
from jax.experimental import pallas as pl

_ROWS = 8


def _body(x_ref, probs_ref, tok_ref):
    x = x_ref[...]
    r, v = x.shape
    m = jnp.max(x, axis=-1, keepdims=True)
    e = jnp.exp(x - m)
    s = jnp.sum(e, axis=-1, keepdims=True)
    probs_ref[...] = e * (1.0 / s)
    idx = lax.broadcasted_iota(jnp.int32, (r, v), 1)
    cand = jnp.where(x == m, idx, v)
    tok_ref[...] = jnp.min(cand, axis=-1, keepdims=True)


@jax.jit
def kernel(logits):
    b, l, v = logits.shape
    last = logits[:, l - 1] + logits[0, 0, 0] * 0.0
    grid = (b // _ROWS,)
    probs, tok = pl.pallas_call(
        _body,
        grid=grid,
        in_specs=[pl.BlockSpec((_ROWS, v), lambda i: (i, 0))],
        out_specs=[
            pl.BlockSpec((_ROWS, v), lambda i: (i, 0)),
            pl.BlockSpec((_ROWS, 1), lambda i: (i, 0)),
        ],
        out_shape=[
            jax.ShapeDtypeStruct((b, v), jnp.float32),
            jax.ShapeDtypeStruct((b, 1), jnp.int32),
        ],
    )(last)
    return (tok.reshape(b), probs)


# speedup vs baseline: 1.2302x; 1.2302x over previous
import jax
import jax.numpy as jnp
from jax import lax
from jax.experimental import pallas as pl
from jax.experimental.pallas import tpu as pltpu

_ROWS = 8
_DEPTH = 4


def _body(x_hbm, probs_hbm, tok_ref, inbuf, outbuf, insem, outsem):
    nblk = x_hbm.shape[0] // _ROWS

    def in_copy(blk, slot):
        return pltpu.make_async_copy(
            x_hbm.at[pl.ds(blk * _ROWS, _ROWS), :], inbuf.at[slot],
            insem.at[slot])

    def out_copy(blk, slot):
        return pltpu.make_async_copy(
            outbuf.at[slot], probs_hbm.at[pl.ds(blk * _ROWS, _ROWS), :],
            outsem.at[slot])

    for p in range(_DEPTH):
        in_copy(p, p).start()

    def step(i, _):
        slot = lax.rem(i, _DEPTH)

        @pl.when(i >= _DEPTH)
        def _():
            out_copy(i - _DEPTH, slot).wait()

        in_copy(i, slot).wait()

        x = inbuf[slot]
        r, v = x.shape
        m = jnp.max(x, axis=-1, keepdims=True)
        e = jnp.exp(x - m)
        s = jnp.sum(e, axis=-1, keepdims=True)
        outbuf[slot] = e * (1.0 / s)
        idx = lax.broadcasted_iota(jnp.int32, (r, v), 1)
        cand = jnp.where(x == m, idx, v)
        tok_ref[pl.ds(i * _ROWS, _ROWS), :] = jnp.min(
            cand, axis=-1, keepdims=True)

        out_copy(i, slot).start()

        @pl.when(i + _DEPTH < nblk)
        def _():
            in_copy(i + _DEPTH, slot).start()

        return 0

    lax.fori_loop(0, nblk, step, 0)

    def drain(i, _):
        slot = lax.rem(i, _DEPTH)
        out_copy(i, slot).wait()
        return 0

    lax.fori_loop(nblk - _DEPTH, nblk, drain, 0)


@jax.jit
def kernel(logits):
    b, l, v = logits.shape
    last = logits[:, l - 1]
    probs, tok = pl.pallas_call(
        _body,
        in_specs=[pl.BlockSpec(memory_space=pltpu.MemorySpace.HBM)],
        out_specs=[
            pl.BlockSpec(memory_space=pltpu.MemorySpace.HBM),
            pl.BlockSpec(memory_space=pltpu.MemorySpace.VMEM),
        ],
        out_shape=[
            jax.ShapeDtypeStruct((b, v), jnp.float32),
            jax.ShapeDtypeStruct((b, 1), jnp.int32),
        ],
        scratch_shapes=[
            pltpu.VMEM((_DEPTH, _ROWS, v), jnp.float32),
            pltpu.VMEM((_DEPTH, _ROWS, v), jnp.float32),
            pltpu.SemaphoreType.DMA((_DEPTH,)),
            pltpu.SemaphoreType.DMA((_DEPTH,)),
        ],
    )(last)
    return (tok.reshape(b), probs)
